# baseline (speedup 1.0000x reference)
import jax
import jax.numpy as jnp
from jax.experimental import pallas as pl
from jax.experimental.pallas import tpu as pltpu

_NUM_ROWS = 1000
_ROW = 8192
_SUB = 64
_LANE = 128
_BATCH = 1024
_BB = 256
_NB = _BATCH // _BB
_CH = 40
_NC = _NUM_ROWS // _CH


def _fused_kernel(labels_ref, w_ref, v0_ref, tensor_ref, out_ref,
                  t3_ref, t2acc, n1acc, inv_ref):
    t = pl.program_id(0)

    @pl.when(t < _NC)
    def _table_phase():
        k = t
        w = w_ref[...]
        wb = w.astype(jnp.bfloat16)
        t3_ref[pl.ds(k * _CH, _CH)] = wb.reshape(_CH, _SUB, _LANE)

        v0b = v0_ref[...].astype(jnp.bfloat16)
        t1 = jax.lax.dot_general(
            v0b, wb, (((1,), (1,)), ((), ())),
            preferred_element_type=jnp.float32)
        t2p = jax.lax.dot_general(
            t1.astype(jnp.bfloat16), wb, (((1,), (0,)), ((), ())),
            preferred_element_type=jnp.float32)
        n1p = jnp.sum(t1 * t1)

        @pl.when(k == 0)
        def _():
            t2acc[...] = jnp.zeros_like(t2acc)
            n1acc[0] = 0.0

        t2acc[...] += t2p
        n1acc[0] += n1p

        @pl.when(k == _NC - 1)
        def _():
            t2 = t2acc[...]
            inv_ref[0] = jnp.sqrt(n1acc[0] / jnp.sum(t2 * t2))

    @pl.when(t >= _NC)
    def _batch_phase():
        i = t - _NC
        inv = inv_ref[0]

        def body(j, _):
            lab = labels_ref[i * _BB + j]
            row = t3_ref[pl.ds(lab, 1)].reshape(1, 8, 8, _LANE)
            out_ref[pl.ds(j, 1)] = (
                tensor_ref[pl.ds(j, 1)] + row.astype(jnp.float32) * inv)
            return 0

        jax.lax.fori_loop(0, _BB, body, 0, unroll=True)


def kernel(tensor, labels, embed_table, v0):
    labels = labels.astype(jnp.int32)

    out = pl.pallas_call(
        _fused_kernel,
        grid=(_NC + _NB,),
        in_specs=[
            pl.BlockSpec(memory_space=pltpu.SMEM),
            pl.BlockSpec((_CH, _ROW),
                         lambda t: (jnp.minimum(t, _NC - 1), 0)),
            pl.BlockSpec(memory_space=pltpu.VMEM),
            pl.BlockSpec((_BB, 8, 8, _LANE),
                         lambda t: (jnp.maximum(t - _NC, 0), 0, 0, 0)),
        ],
        out_specs=pl.BlockSpec((_BB, 8, 8, _LANE),
                               lambda t: (jnp.maximum(t - _NC, 0), 0, 0, 0)),
        out_shape=jax.ShapeDtypeStruct(tensor.shape, jnp.float32),
        scratch_shapes=[
            pltpu.VMEM((_NUM_ROWS, _SUB, _LANE), jnp.bfloat16),
            pltpu.VMEM((1, _ROW), jnp.float32),
            pltpu.SMEM((1,), jnp.float32),
            pltpu.SMEM((1,), jnp.float32),
        ],
    )(labels, embed_table, v0.reshape(1, _ROW), tensor)

    return out


# speedup vs baseline: 1.2006x; 1.2006x over previous
import jax
import jax.numpy as jnp
from jax.experimental import pallas as pl
from jax.experimental.pallas import tpu as pltpu

_NUM_ROWS = 1000
_ROW = 8192
_SUB = 64
_LANE = 128
_BATCH = 1024
_BB = 128
_NB = _BATCH // _BB
_CH = 200
_NC = _NUM_ROWS // _CH


def _fused_kernel(labels_ref, w_ref, v0_ref, tensor_ref, out_ref,
                  t3_ref, t2acc, n1acc, inv_ref):
    t = pl.program_id(0)

    @pl.when(t < _NC)
    def _table_phase():
        k = t
        w = w_ref[...]
        wb = w.astype(jnp.bfloat16)
        t3_ref[pl.ds(k * _CH, _CH)] = wb.reshape(_CH, _SUB, _LANE)

        v0b = v0_ref[...].astype(jnp.bfloat16)
        t1 = jax.lax.dot_general(
            v0b, wb, (((1,), (1,)), ((), ())),
            preferred_element_type=jnp.float32)
        t2p = jax.lax.dot_general(
            t1.astype(jnp.bfloat16), wb, (((1,), (0,)), ((), ())),
            preferred_element_type=jnp.float32)
        n1p = jnp.sum(t1 * t1)

        @pl.when(k == 0)
        def _():
            t2acc[...] = jnp.zeros_like(t2acc)
            n1acc[0] = 0.0

        t2acc[...] += t2p
        n1acc[0] += n1p

        @pl.when(k == _NC - 1)
        def _():
            t2 = t2acc[...]
            inv_ref[0] = jnp.sqrt(n1acc[0] / jnp.sum(t2 * t2))

    @pl.when(t >= _NC)
    def _batch_phase():
        i = t - _NC
        inv = inv_ref[0]

        def body(j, _):
            lab = labels_ref[i * _BB + j]
            row = t3_ref[pl.ds(lab, 1)].reshape(1, 8, 8, _LANE)
            out_ref[pl.ds(j, 1)] = (
                tensor_ref[pl.ds(j, 1)] + row.astype(jnp.float32) * inv)
            return 0

        jax.lax.fori_loop(0, _BB, body, 0, unroll=True)


def kernel(tensor, labels, embed_table, v0):
    labels = labels.astype(jnp.int32)

    out = pl.pallas_call(
        _fused_kernel,
        grid=(_NC + _NB,),
        in_specs=[
            pl.BlockSpec(memory_space=pltpu.SMEM),
            pl.BlockSpec((_CH, _ROW),
                         lambda t: (jnp.minimum(t, _NC - 1), 0)),
            pl.BlockSpec(memory_space=pltpu.VMEM),
            pl.BlockSpec((_BB, 8, 8, _LANE),
                         lambda t: (jnp.maximum(t - _NC, 0), 0, 0, 0)),
        ],
        out_specs=pl.BlockSpec((_BB, 8, 8, _LANE),
                               lambda t: (jnp.maximum(t - _NC, 0), 0, 0, 0)),
        out_shape=jax.ShapeDtypeStruct(tensor.shape, jnp.float32),
        scratch_shapes=[
            pltpu.VMEM((_NUM_ROWS, _SUB, _LANE), jnp.bfloat16),
            pltpu.VMEM((1, _ROW), jnp.float32),
            pltpu.SMEM((1,), jnp.float32),
            pltpu.SMEM((1,), jnp.float32),
        ],
    )(labels, embed_table, v0.reshape(1, _ROW), tensor)

    return out


# speedup vs baseline: 1.2535x; 1.0441x over previous
import jax
import jax.numpy as jnp
from jax.experimental import pallas as pl
from jax.experimental.pallas import tpu as pltpu

_NUM_ROWS = 1000
_ROW = 8192
_SUB = 64
_LANE = 128
_BATCH = 1024
_BB = 128
_NB = _BATCH // _BB
_CH = 200
_NC = _NUM_ROWS // _CH


def _fused_kernel(labels_ref, w_ref, v0_ref, tensor_ref, out_ref,
                  t3_ref, t2acc, n1acc, inv_ref):
    t = pl.program_id(0)

    @pl.when(t < _NC)
    def _table_phase():
        k = t
        w = w_ref[...]
        wb = w.astype(jnp.bfloat16)
        t3_ref[pl.ds(k * _CH, _CH)] = wb.reshape(_CH, _SUB, _LANE)

        v0f = v0_ref[...]
        t1 = jax.lax.dot_general(
            w, v0f, (((1,), (1,)), ((), ())),
            preferred_element_type=jnp.float32)
        t2p = jax.lax.dot_general(
            t1.astype(jnp.bfloat16), wb, (((0,), (0,)), ((), ())),
            preferred_element_type=jnp.float32)
        n1p = jnp.sum(t1 * t1)

        @pl.when(k == 0)
        def _():
            t2acc[...] = jnp.zeros_like(t2acc)
            n1acc[0] = 0.0

        t2acc[...] += t2p
        n1acc[0] += n1p

        @pl.when(k == _NC - 1)
        def _():
            t2 = t2acc[...]
            inv_ref[0] = jnp.sqrt(n1acc[0] / jnp.sum(t2 * t2))

    @pl.when(t >= _NC)
    def _batch_phase():
        i = t - _NC
        inv = inv_ref[0]

        def body(j, _):
            lab = labels_ref[i * _BB + j]
            row = t3_ref[pl.ds(lab, 1)].reshape(1, 8, 8, _LANE)
            out_ref[pl.ds(j, 1)] = (
                tensor_ref[pl.ds(j, 1)] + row.astype(jnp.float32) * inv)
            return 0

        jax.lax.fori_loop(0, _BB, body, 0, unroll=True)


def kernel(tensor, labels, embed_table, v0):
    labels = labels.astype(jnp.int32)

    out = pl.pallas_call(
        _fused_kernel,
        grid=(_NC + _NB,),
        in_specs=[
            pl.BlockSpec(memory_space=pltpu.SMEM),
            pl.BlockSpec((_CH, _ROW),
                         lambda t: (jnp.minimum(t, _NC - 1), 0)),
            pl.BlockSpec(memory_space=pltpu.VMEM),
            pl.BlockSpec((_BB, 8, 8, _LANE),
                         lambda t: (jnp.maximum(t - _NC, 0), 0, 0, 0)),
        ],
        out_specs=pl.BlockSpec((_BB, 8, 8, _LANE),
                               lambda t: (jnp.maximum(t - _NC, 0), 0, 0, 0)),
        out_shape=jax.ShapeDtypeStruct(tensor.shape, jnp.float32),
        scratch_shapes=[
            pltpu.VMEM((_NUM_ROWS, _SUB, _LANE), jnp.bfloat16),
            pltpu.VMEM((1, _ROW), jnp.float32),
            pltpu.SMEM((1,), jnp.float32),
            pltpu.SMEM((1,), jnp.float32),
        ],
    )(labels, embed_table, v0.reshape(1, _ROW), tensor)

    return out
